# baseline (speedup 1.0000x reference)
import functools

import jax
import jax.numpy as jnp
from jax import lax
from jax.experimental import pallas as pl
from jax.experimental.pallas import tpu as pltpu
from jax.experimental.pallas import tpu_sc as plsc

_N2 = 8388608
_NW = 32
_PW = _N2 // _NW
_CH = 16384
_NCHUNK = _PW // _CH

_GDN = lax.GatherDimensionNumbers(
    offset_dims=(), collapsed_slice_dims=(0,), start_index_map=(0,))


def _pair_swap(v, swap_idx):
    return lax.gather(v, swap_idx[:, None], _GDN, slice_sizes=(1,),
                      mode=lax.GatherScatterMode.PROMISE_IN_BOUNDS)


def _sc_partials(pred_flat, true_flat):
    mesh = plsc.VectorSubcoreMesh(core_axis_name="c", subcore_axis_name="s")

    @functools.partial(
        pl.kernel,
        mesh=mesh,
        out_type=jax.ShapeDtypeStruct((_NW, 16), jnp.float32),
        scratch_types=[
            pltpu.VMEM((_CH,), jnp.float32),
            pltpu.VMEM((_CH,), jnp.float32),
            pltpu.VMEM((16,), jnp.float32),
        ],
    )
    def k(pred_hbm, true_hbm, out_hbm, pbuf, tbuf, obuf):
        cid = lax.axis_index("c")
        sid = lax.axis_index("s")
        wid = sid * 2 + cid
        base = wid * _PW

        lane = lax.iota(jnp.int32, 16)
        swap_idx = lane ^ 1
        is_col0 = (lane & 1) == 0

        def chunk_body(ci, acc):
            off = base + ci * _CH
            pltpu.sync_copy(pred_hbm.at[pl.ds(off, _CH)], pbuf)
            pltpu.sync_copy(true_hbm.at[pl.ds(off, _CH)], tbuf)

            def pos_body(j, a):
                p = pbuf[pl.ds(j * 16, 16)]
                t = tbuf[pl.ds(j * 16, 16)]
                t_nbr = _pair_swap(t, swap_idx)
                invisible = is_col0 & (t_nbr == 0.0)
                d = jnp.where(invisible, 0.5 * p - 1.5 * t, p - t)
                return a + d * d

            return lax.fori_loop(0, _CH // 16, pos_body, acc)

        acc = lax.fori_loop(
            0, _NCHUNK, chunk_body, jnp.zeros((16,), jnp.float32))
        obuf[...] = acc
        pltpu.sync_copy(obuf, out_hbm.at[wid])

    return k(pred_flat, true_flat)


def kernel(pred, true):
    partials = _sc_partials(pred.reshape(-1), true.reshape(-1))
    return jnp.sum(partials) / jnp.float32(_N2)


# speedup vs baseline: 120.3029x; 120.3029x over previous
import functools

import jax
import jax.numpy as jnp
from jax import lax
from jax.experimental import pallas as pl
from jax.experimental.pallas import tpu as pltpu
from jax.experimental.pallas import tpu_sc as plsc

_N = 4194304
_N2 = 2 * _N
_NB = _N // 128
_NW = 32
_PW = _N2 // _NW
_CH = 16384
_NCHUNK = _PW // _CH
_BLK = 256


def _sc_partials(pred_flat, true_flat):
    mesh = plsc.VectorSubcoreMesh(core_axis_name="c", subcore_axis_name="s")

    @functools.partial(
        pl.kernel,
        mesh=mesh,
        out_type=jax.ShapeDtypeStruct((_NW, 16), jnp.float32),
        scratch_types=[
            pltpu.VMEM((_CH,), jnp.float32),
            pltpu.VMEM((_CH,), jnp.float32),
            pltpu.VMEM((16,), jnp.float32),
        ],
    )
    def k(pred_hbm, true_hbm, out_hbm, pbuf, tbuf, obuf):
        cid = lax.axis_index("c")
        sid = lax.axis_index("s")
        wid = sid * 2 + cid
        base = wid * _PW

        def chunk_body(ci, accs):
            off = base + ci * _CH
            pltpu.sync_copy(pred_hbm.at[pl.ds(off, _CH)], pbuf)
            pltpu.sync_copy(true_hbm.at[pl.ds(off, _CH)], tbuf)

            def blk_body(g, a):
                o = g * _BLK
                a0, a1 = a
                for v in range(0, 128, 16):
                    p0 = pbuf[pl.ds(o + v, 16)]
                    t0 = tbuf[pl.ds(o + v, 16)]
                    p1 = pbuf[pl.ds(o + 128 + v, 16)]
                    t1 = tbuf[pl.ds(o + 128 + v, 16)]
                    d0 = jnp.where(t1 == 0.0, 0.5 * p0 - 1.5 * t0, p0 - t0)
                    d1 = p1 - t1
                    a0 = a0 + d0 * d0
                    a1 = a1 + d1 * d1
                return (a0, a1)

            return lax.fori_loop(0, _CH // _BLK, blk_body, accs)

        zero = jnp.zeros((16,), jnp.float32)
        acc0, acc1 = lax.fori_loop(0, _NCHUNK, chunk_body, (zero, zero))
        obuf[...] = acc0 + acc1
        pltpu.sync_copy(obuf, out_hbm.at[wid])

    return k(pred_flat, true_flat)


def _native_flat(x):
    return x.reshape(_NB, 128, 2).transpose(0, 2, 1).reshape(-1)


def kernel(pred, true):
    partials = _sc_partials(_native_flat(pred), _native_flat(true))
    return jnp.sum(partials) / jnp.float32(_N2)


# speedup vs baseline: 193.7742x; 1.6107x over previous
import functools

import jax
import jax.numpy as jnp
from jax import lax
from jax.experimental import pallas as pl
from jax.experimental.pallas import tpu as pltpu
from jax.experimental.pallas import tpu_sc as plsc

_N = 4194304
_N2 = 2 * _N
_NB = _N // 128
_NW = 32
_PW = _N2 // _NW
_CH = 16384
_NCHUNK = _PW // _CH
_BLK = 256


def _sc_partials(pred_flat, true_flat):
    mesh = plsc.VectorSubcoreMesh(core_axis_name="c", subcore_axis_name="s")

    @functools.partial(
        pl.kernel,
        mesh=mesh,
        out_type=jax.ShapeDtypeStruct((_NW, 16), jnp.float32),
        scratch_types=[
            pltpu.VMEM((_CH,), jnp.float32),
            pltpu.VMEM((_CH,), jnp.float32),
            pltpu.VMEM((_CH,), jnp.float32),
            pltpu.VMEM((_CH,), jnp.float32),
            pltpu.VMEM((16,), jnp.float32),
            pltpu.SemaphoreType.DMA,
            pltpu.SemaphoreType.DMA,
        ],
    )
    def k(pred_hbm, true_hbm, out_hbm, pb0, tb0, pb1, tb1, obuf, sem0, sem1):
        cid = lax.axis_index("c")
        sid = lax.axis_index("s")
        wid = sid * 2 + cid
        base = wid * _PW

        def start(ci, pb, tb, sem):
            off = base + ci * _CH
            pltpu.async_copy(pred_hbm.at[pl.ds(off, _CH)], pb, sem)
            pltpu.async_copy(true_hbm.at[pl.ds(off, _CH)], tb, sem)

        def drain(pb, tb, sem):
            pltpu.make_async_copy(pred_hbm.at[pl.ds(0, _CH)], pb, sem).wait()
            pltpu.make_async_copy(true_hbm.at[pl.ds(0, _CH)], tb, sem).wait()

        def compute(pb, tb, accs):
            def blk_body(g, a):
                o = g * _BLK
                a0, a1 = a
                for v in range(0, 128, 16):
                    p0 = pb[pl.ds(o + v, 16)]
                    t0 = tb[pl.ds(o + v, 16)]
                    p1 = pb[pl.ds(o + 128 + v, 16)]
                    t1 = tb[pl.ds(o + 128 + v, 16)]
                    d0 = jnp.where(t1 == 0.0, 0.5 * p0 - 1.5 * t0, p0 - t0)
                    d1 = p1 - t1
                    a0 = a0 + d0 * d0
                    a1 = a1 + d1 * d1
                return (a0, a1)

            return lax.fori_loop(0, _CH // _BLK, blk_body, accs)

        nstep = _NCHUNK // 2
        start(0, pb0, tb0, sem0)
        start(1, pb1, tb1, sem1)

        def step(s, accs):
            drain(pb0, tb0, sem0)
            accs = compute(pb0, tb0, accs)

            @pl.when(s < nstep - 1)
            def _():
                start(2 * s + 2, pb0, tb0, sem0)

            drain(pb1, tb1, sem1)
            accs = compute(pb1, tb1, accs)

            @pl.when(s < nstep - 1)
            def _():
                start(2 * s + 3, pb1, tb1, sem1)

            return accs

        zero = jnp.zeros((16,), jnp.float32)
        acc0, acc1 = lax.fori_loop(0, nstep, step, (zero, zero))
        obuf[...] = acc0 + acc1
        pltpu.sync_copy(obuf, out_hbm.at[wid])

    return k(pred_flat, true_flat)


def _native_flat(x):
    return x.reshape(_NB, 128, 2).transpose(0, 2, 1).reshape(-1)


def kernel(pred, true):
    partials = _sc_partials(_native_flat(pred), _native_flat(true))
    return jnp.sum(partials) / jnp.float32(_N2)
